# baseline (speedup 1.0000x reference)
import functools

import jax
import jax.numpy as jnp
from jax import lax
from jax.experimental import pallas as pl
from jax.experimental.pallas import tpu as pltpu
from jax.experimental.pallas import tpu_sc as plsc

NC = 2
NS = 16
NW = NC * NS
B = 16384
D = 64
BPW = B // NW
CHUNK = 64


def _gather_table(tab_hbm, idx_v, emb_v, sem):
    copies = [
        pltpu.async_copy(tab_hbm.at[c].at[idx_v], emb_v.at[c], sem)
        for c in range(D)
    ]
    for cp in copies:
        cp.wait()


def _gather_body(users_hbm, isbns_hbm, utab_hbm, itab_hbm,
                 uout_hbm, iout_hbm, uidx_v, iidx_v, emb_v, sem):
    wid = lax.axis_index("s") * NC + lax.axis_index("c")
    base = wid * BPW
    pltpu.sync_copy(users_hbm.at[pl.ds(base, BPW)], uidx_v)
    pltpu.sync_copy(isbns_hbm.at[pl.ds(base, BPW)], iidx_v)
    _gather_table(utab_hbm, uidx_v, emb_v, sem)
    pltpu.sync_copy(emb_v, uout_hbm.at[wid])
    _gather_table(itab_hbm, iidx_v, emb_v, sem)
    pltpu.sync_copy(emb_v, iout_hbm.at[wid])


def _sc_gather(users, isbns, user_table, isbn_table):
    mesh = plsc.VectorSubcoreMesh(core_axis_name="c", subcore_axis_name="s")
    f = pl.kernel(
        _gather_body,
        out_type=(
            jax.ShapeDtypeStruct((NW, D, BPW), jnp.float32),
            jax.ShapeDtypeStruct((NW, D, BPW), jnp.float32),
        ),
        mesh=mesh,
        compiler_params=pltpu.CompilerParams(use_tc_tiling_on_sc=False),
        scratch_types=[
            pltpu.VMEM((BPW,), jnp.int32),
            pltpu.VMEM((BPW,), jnp.int32),
            pltpu.VMEM((D, BPW), jnp.float32),
            pltpu.SemaphoreType.DMA,
        ],
    )
    return f(users, isbns, user_table.T, isbn_table.T)


BM = 1024


def _mlp_body(u_ref, i_ref, w1u_ref, w1i_ref, b1_ref, w2_ref, b2_ref,
              w3_ref, b3_ref, o_ref):
    xu = jnp.transpose(u_ref[0])
    xi = jnp.transpose(i_ref[0])
    x = jnp.dot(xu, w1u_ref[...], preferred_element_type=jnp.float32)
    x = x + jnp.dot(xi, w1i_ref[...],
                    preferred_element_type=jnp.float32)
    x = jnp.maximum(x + b1_ref[...], 0.0)
    x = jnp.maximum(
        jnp.dot(x, w2_ref[...], preferred_element_type=jnp.float32)
        + b2_ref[...], 0.0)
    o_ref[...] = (jnp.dot(x, w3_ref[...], preferred_element_type=jnp.float32)
                  + b3_ref[...])


def _tc_mlp(u_emb, i_emb, W1, b1, W2, b2, W3, b3):
    full = lambda s: pl.BlockSpec(s, lambda m: (0, 0))
    return pl.pallas_call(
        _mlp_body,
        grid=(NW,),
        in_specs=[
            pl.BlockSpec((1, D, BPW), lambda m: (m, 0, 0)),
            pl.BlockSpec((1, D, BPW), lambda m: (m, 0, 0)),
            full((D, 64)),
            full((D, 64)),
            full((1, 64)),
            full((64, 32)),
            full((1, 32)),
            full((32, 1)),
            full((1, 1)),
        ],
        out_specs=pl.BlockSpec((BPW, 1), lambda m: (m, 0)),
        out_shape=jax.ShapeDtypeStruct((B, 1), jnp.float32),
    )(u_emb, i_emb,
      W1[:D], W1[D:], b1.reshape(1, 64), W2, b2.reshape(1, 32),
      W3, b3.reshape(1, 1))


def kernel(users, isbns, user_table, isbn_table, W1, b1, W2, b2, W3, b3):
    u_emb, i_emb = _sc_gather(users, isbns, user_table, isbn_table)
    return _tc_mlp(u_emb, i_emb, W1, b1, W2, b2, W3, b3)


# speedup vs baseline: 17.2284x; 17.2284x over previous
import functools

import jax
import jax.numpy as jnp
from jax import lax
from jax.experimental import pallas as pl
from jax.experimental.pallas import tpu as pltpu
from jax.experimental.pallas import tpu_sc as plsc

NC = 2
NS = 16
NW = NC * NS
B = 16384
D = 64
BPW = B // NW
GB = 128
NCH = BPW // GB
CW = 12800
KU = 512000
KI = 51200


def _relayout_body(a_ref, b_ref, o_ref):
    eye = (lax.broadcasted_iota(jnp.int32, (D, D), 0) ==
           lax.broadcasted_iota(jnp.int32, (D, D), 1)).astype(jnp.float32)
    lo = lax.dot_general(a_ref[...], eye, (((0,), (0,)), ((), ())),
                         preferred_element_type=jnp.float32)
    hi = lax.dot_general(b_ref[...], eye, (((0,), (0,)), ((), ())),
                         preferred_element_type=jnp.float32)
    o_ref[...] = jnp.concatenate([lo, hi], axis=1)


def _relayout(tabT, k):
    nb = k // CW
    last = (tabT.shape[1] - 1) // CW
    return pl.pallas_call(
        _relayout_body,
        grid=(nb,),
        in_specs=[
            pl.BlockSpec((D, CW), lambda m: (0, m)),
            pl.BlockSpec((D, CW), lambda m: (0, jnp.minimum(m + nb, last))),
        ],
        out_specs=pl.BlockSpec((CW, 2 * D), lambda m: (m, 0)),
        out_shape=jax.ShapeDtypeStruct((k, 2 * D), jnp.float32),
    )(tabT, tabT)


def _gather_body(users_hbm, isbns_hbm, utab_hbm, itab_hbm,
                 uout_hbm, iout_hbm,
                 uidx_v, iidx_v, ugi_v, igi_v, pair_v, usem, isem):
    wid = lax.axis_index("s") * NC + lax.axis_index("c")
    base = wid * BPW
    pltpu.sync_copy(users_hbm.at[pl.ds(base, BPW)], uidx_v)
    pltpu.sync_copy(isbns_hbm.at[pl.ds(base, BPW)], iidx_v)
    for k in range(BPW // 16):
        c, l = divmod(k, 8)
        u = uidx_v[pl.ds(k * 16, 16)]
        ugi_v[c, pl.ds(l * 16, 16)] = jnp.where(u >= KU, u - KU, u)
        i = iidx_v[pl.ds(k * 16, 16)]
        igi_v[c, pl.ds(l * 16, 16)] = jnp.where(i >= KI, i - KI, i)

    ucopies = [pltpu.async_copy(utab_hbm.at[ugi_v.at[c]],
                                pair_v.at[pl.ds(c * GB, GB)], usem)
               for c in range(NCH)]
    for cp in ucopies:
        cp.wait()
    pltpu.sync_copy(pair_v, uout_hbm.at[pl.ds(base, BPW)])
    icopies = [pltpu.async_copy(itab_hbm.at[igi_v.at[c]],
                                pair_v.at[pl.ds(c * GB, GB)], isem)
               for c in range(NCH)]
    for cp in icopies:
        cp.wait()
    pltpu.sync_copy(pair_v, iout_hbm.at[pl.ds(base, BPW)])


def _sc_gather(users, isbns, upacked, ipacked):
    mesh = plsc.VectorSubcoreMesh(core_axis_name="c", subcore_axis_name="s")
    f = pl.kernel(
        _gather_body,
        out_type=(
            jax.ShapeDtypeStruct((B, 2 * D), jnp.float32),
            jax.ShapeDtypeStruct((B, 2 * D), jnp.float32),
        ),
        mesh=mesh,
        scratch_types=[
            pltpu.VMEM((BPW,), jnp.int32),
            pltpu.VMEM((BPW,), jnp.int32),
            pltpu.VMEM((NCH, GB), jnp.int32),
            pltpu.VMEM((NCH, GB), jnp.int32),
            pltpu.VMEM((BPW, 2 * D), jnp.float32),
            pltpu.SemaphoreType.DMA,
            pltpu.SemaphoreType.DMA,
        ],
    )
    return f(users, isbns, upacked, ipacked)


BM = 2048


def _mlp_body(up_ref, ip_ref, uhi_ref, ihi_ref, w1u_ref, w1i_ref, b1_ref,
              w2_ref, b2_ref, w3_ref, b3_ref, o_ref):
    up = up_ref[...]
    ip = ip_ref[...]
    xu = jnp.where(uhi_ref[...] > 0, up[:, D:], up[:, :D])
    xi = jnp.where(ihi_ref[...] > 0, ip[:, D:], ip[:, :D])
    x = jnp.dot(xu, w1u_ref[...], preferred_element_type=jnp.float32)
    x = x + jnp.dot(xi, w1i_ref[...], preferred_element_type=jnp.float32)
    x = jnp.maximum(x + b1_ref[...], 0.0)
    x = jnp.maximum(
        jnp.dot(x, w2_ref[...], preferred_element_type=jnp.float32)
        + b2_ref[...], 0.0)
    o_ref[...] = (jnp.dot(x, w3_ref[...], preferred_element_type=jnp.float32)
                  + b3_ref[...])


def _tc_mlp(upairs, ipairs, uhi, ihi, W1, b1, W2, b2, W3, b3):
    full = lambda s: pl.BlockSpec(s, lambda m: (0, 0))
    return pl.pallas_call(
        _mlp_body,
        grid=(B // BM,),
        in_specs=[
            pl.BlockSpec((BM, 2 * D), lambda m: (m, 0)),
            pl.BlockSpec((BM, 2 * D), lambda m: (m, 0)),
            pl.BlockSpec((BM, 1), lambda m: (m, 0)),
            pl.BlockSpec((BM, 1), lambda m: (m, 0)),
            full((D, 64)),
            full((D, 64)),
            full((1, 64)),
            full((64, 32)),
            full((1, 32)),
            full((32, 1)),
            full((1, 1)),
        ],
        out_specs=pl.BlockSpec((BM, 1), lambda m: (m, 0)),
        out_shape=jax.ShapeDtypeStruct((B, 1), jnp.float32),
    )(upairs, ipairs, uhi, ihi,
      W1[:D], W1[D:], b1.reshape(1, 64), W2, b2.reshape(1, 32),
      W3, b3.reshape(1, 1))


def kernel(users, isbns, user_table, isbn_table, W1, b1, W2, b2, W3, b3):
    upacked = _relayout(user_table.T, KU)
    ipacked = _relayout(isbn_table.T, KI)
    upairs, ipairs = _sc_gather(users, isbns, upacked, ipacked)
    uhi = (users >= KU).astype(jnp.float32).reshape(B, 1)
    ihi = (isbns >= KI).astype(jnp.float32).reshape(B, 1)
    return _tc_mlp(upairs, ipairs, uhi, ihi, W1, b1, W2, b2, W3, b3)
